# baseline (speedup 1.0000x reference)
import functools

import jax
import jax.numpy as jnp
from jax import lax
from jax.experimental import pallas as pl
from jax.experimental.pallas import tpu as pltpu
from jax.experimental.pallas import tpu_sc as plsc

_NC = 2
_NS = 16
_NW = _NC * _NS
_L = 16
_NQ = 4


def _fused(emb_table, text_t, fc_w, fc_b_tiled, num_class):
    hist, batch = text_t.shape
    vocab, dim = emb_table.shape
    b_per_w = batch // _NW
    n_slice = dim // _L
    rows_per_blk = _L // num_class
    q_rows = b_per_w // _NQ
    q_blks = q_rows // rows_per_blk
    inv_len = 1.0 / hist

    mesh = plsc.VectorSubcoreMesh(
        core_axis_name="c", subcore_axis_name="s",
        num_cores=_NC, num_subcores=_NS)

    @functools.partial(
        pl.kernel,
        out_type=jax.ShapeDtypeStruct((batch * num_class,), jnp.float32),
        mesh=mesh,
        scratch_types=[
            pltpu.VMEM((hist, b_per_w), jnp.int32),
            pltpu.VMEM((b_per_w, dim), jnp.float32),
            pltpu.VMEM((num_class, dim), jnp.float32),
            pltpu.VMEM((_L,), jnp.float32),
            pltpu.VMEM((b_per_w * num_class,), jnp.float32),
            pltpu.SemaphoreType.DMA,
            [pltpu.SemaphoreType.DMA] * _NQ,
        ],
    )
    def k(table_hbm, textt_hbm, w_hbm, b_hbm, out_hbm,
          idx_v, acc_v, w_v, b_v, out_v, idx_sem, sems):
        wid = lax.axis_index("s") * _NC + lax.axis_index("c")
        base = wid * b_per_w
        idx_cp = pltpu.async_copy(
            textt_hbm.at[:, pl.ds(base, b_per_w)], idx_v, idx_sem)

        zero = jnp.zeros((_L,), jnp.float32)

        def zloop(r, carry):
            for s in range(n_slice):
                acc_v[r, pl.ds(s * _L, _L)] = zero
            return carry

        lax.fori_loop(0, b_per_w, zloop, 0)
        idx_cp.wait()

        for q in range(_NQ):
            r0 = q * q_rows

            def fire(j, carry, r0=r0, sem=sems[q]):
                pltpu.async_copy(
                    table_hbm.at[idx_v.at[j, pl.ds(r0, q_rows)]],
                    acc_v.at[pl.ds(r0, q_rows)], sem, add=True)
                return carry

            lax.fori_loop(0, hist, fire, 0)

        pltpu.sync_copy(w_hbm, w_v)
        pltpu.sync_copy(b_hbm, b_v)

        lane = lax.iota(jnp.int32, _L)
        lane_eq = [lane == j for j in range(_L)]
        row_of_lane = [
            (lane >= i * num_class) & (lane < (i + 1) * num_class)
            for i in range(rows_per_blk - 1)
        ]
        perms = [lane ^ sh for sh in (1, 2, 4, 8)]
        dnums = lax.GatherDimensionNumbers(
            offset_dims=(), collapsed_slice_dims=(0,), start_index_map=(0,))

        def lane_perm(v, p):
            return lax.gather(
                v, p[:, None], dimension_numbers=dnums, slice_sizes=(1,),
                mode=lax.GatherScatterMode.PROMISE_IN_BOUNDS)

        def block(blk, carry):
            r0 = blk * rows_per_blk
            logit_vecs = []
            maxes = []
            for i in range(rows_per_blk):
                r = r0 + i
                ts = []
                for s in range(n_slice):
                    x = acc_v[r, pl.ds(s * _L, _L)] * inv_len
                    e = jnp.exp(x + x)
                    ts.append(1.0 - 2.0 / (e + 1.0))
                row_vecs = []
                for c in range(num_class):
                    p = ts[0] * w_v[c, pl.ds(0, _L)]
                    for s in range(1, n_slice):
                        p = p + ts[s] * w_v[c, pl.ds(s * _L, _L)]
                    for pm in perms:
                        p = p + lane_perm(p, pm)
                    row_vecs.append(p)
                logit_vecs.append(row_vecs)
                m = row_vecs[0]
                for c in range(1, num_class):
                    m = jnp.maximum(m, row_vecs[c])
                maxes.append(m)

            lvec = logit_vecs[0][0]
            for j in range(1, _L):
                lvec = jnp.where(lane_eq[j],
                                 logit_vecs[j // num_class][j % num_class],
                                 lvec)
            lvec = lvec + b_v[...]
            mvec = maxes[-1]
            for i in range(rows_per_blk - 1):
                mvec = jnp.where(row_of_lane[i], maxes[i], mvec)
            evec = jnp.exp(lvec - mvec)
            svec = evec + lane_perm(evec, perms[0])
            svec = svec + lane_perm(svec, perms[1])
            out_v[pl.ds(blk * _L, _L)] = evec / svec
            return carry

        for q in range(_NQ):

            def drain(j, carry, sem=sems[q]):
                pltpu.make_async_copy(
                    table_hbm.at[idx_v.at[0, pl.ds(0, q_rows)]],
                    acc_v.at[pl.ds(0, q_rows)], sem).wait()
                return carry

            lax.fori_loop(0, hist, drain, 0)
            lax.fori_loop(q * q_blks, (q + 1) * q_blks, block, 0)

        pltpu.sync_copy(
            out_v, out_hbm.at[pl.ds(base * num_class, b_per_w * num_class)])

    return k(emb_table, text_t, fc_w, fc_b_tiled)


def kernel(text, offsets, emb_table, fc_w, fc_b):
    del offsets
    batch, _ = text.shape
    num_class = fc_w.shape[0]
    text_t = text.astype(jnp.int32).T
    fc_b_tiled = jnp.tile(fc_b, _L // num_class)
    flat = _fused(emb_table, text_t, fc_w, fc_b_tiled, num_class)
    return flat.reshape(batch, num_class)


# speedup vs baseline: 1.0606x; 1.0606x over previous
import functools

import jax
import jax.numpy as jnp
from jax import lax
from jax.experimental import pallas as pl
from jax.experimental.pallas import tpu as pltpu
from jax.experimental.pallas import tpu_sc as plsc

_NC = 2
_NS = 16
_NW = _NC * _NS
_L = 16


def _pooled_sum(emb_table, text_t):
    hist, batch = text_t.shape
    vocab, dim = emb_table.shape
    b_per_w = batch // _NW
    n_slice = dim // _L

    mesh = plsc.VectorSubcoreMesh(
        core_axis_name="c", subcore_axis_name="s",
        num_cores=_NC, num_subcores=_NS)

    @functools.partial(
        pl.kernel,
        out_type=jax.ShapeDtypeStruct((batch, dim), jnp.float32),
        mesh=mesh,
        scratch_types=[
            pltpu.VMEM((hist, b_per_w), jnp.int32),
            pltpu.VMEM((b_per_w, dim), jnp.float32),
            pltpu.SemaphoreType.DMA,
            pltpu.SemaphoreType.DMA,
        ],
    )
    def k(table_hbm, textt_hbm, out_hbm, idx_v, acc_v, sem, idx_sem):
        wid = lax.axis_index("s") * _NC + lax.axis_index("c")
        base = wid * b_per_w
        idx_cp = pltpu.async_copy(
            textt_hbm.at[:, pl.ds(base, b_per_w)], idx_v, idx_sem)

        zero = jnp.zeros((_L,), jnp.float32)

        def zloop(r, carry):
            for s in range(n_slice):
                acc_v[r, pl.ds(s * _L, _L)] = zero
            return carry

        lax.fori_loop(0, b_per_w, zloop, 0)
        idx_cp.wait()

        def fire(j, carry):
            pltpu.async_copy(table_hbm.at[idx_v.at[j]], acc_v, sem, add=True)
            return carry

        lax.fori_loop(0, hist, fire, 0)

        def drain(j, carry):
            pltpu.make_async_copy(table_hbm.at[idx_v.at[0]], acc_v, sem).wait()
            return carry

        lax.fori_loop(0, hist, drain, 0)
        pltpu.sync_copy(acc_v, out_hbm.at[pl.ds(base, b_per_w)])

    return k(emb_table, text_t)


def _head_body(x_ref, w_ref, b_ref, o_ref, *, inv_len):
    x = jnp.tanh(x_ref[...] * inv_len)
    logits = lax.dot_general(
        x, w_ref[...], dimension_numbers=(((1,), (1,)), ((), ())),
        preferred_element_type=jnp.float32)
    logits = logits + b_ref[...]
    m = jnp.max(logits, axis=1, keepdims=True)
    e = jnp.exp(logits - m)
    o_ref[...] = e / jnp.sum(e, axis=1, keepdims=True)


def kernel(text, offsets, emb_table, fc_w, fc_b):
    del offsets
    batch, hist = text.shape
    num_class = fc_w.shape[0]
    text_t = text.astype(jnp.int32).T
    pooled = _pooled_sum(emb_table, text_t)
    blk = batch // 8
    head = pl.pallas_call(
        functools.partial(_head_body, inv_len=1.0 / hist),
        grid=(batch // blk,),
        in_specs=[
            pl.BlockSpec((blk, emb_table.shape[1]), lambda i: (i, 0)),
            pl.BlockSpec(fc_w.shape, lambda i: (0, 0)),
            pl.BlockSpec((1, num_class), lambda i: (0, 0)),
        ],
        out_specs=pl.BlockSpec((blk, num_class), lambda i: (i, 0)),
        out_shape=jax.ShapeDtypeStruct((batch, num_class), jnp.float32),
    )
    return head(pooled, fc_w, fc_b.reshape(1, num_class))


# speedup vs baseline: 1.1925x; 1.1244x over previous
import functools

import jax
import jax.numpy as jnp
from jax import lax
from jax.experimental import pallas as pl
from jax.experimental.pallas import tpu as pltpu
from jax.experimental.pallas import tpu_sc as plsc

_NC = 2
_NS = 16
_NW = _NC * _NS
_L = 16


def _pooled_sum(emb_table, text_t):
    hist, batch = text_t.shape
    vocab, dim = emb_table.shape
    b_per_w = batch // _NW
    n_slice = dim // _L

    mesh = plsc.VectorSubcoreMesh(
        core_axis_name="c", subcore_axis_name="s",
        num_cores=_NC, num_subcores=_NS)

    @functools.partial(
        pl.kernel,
        out_type=jax.ShapeDtypeStruct((batch, dim), jnp.float32),
        mesh=mesh,
        scratch_types=[
            pltpu.VMEM((hist, b_per_w), jnp.int32),
            pltpu.VMEM((b_per_w, dim), jnp.float32),
            pltpu.SemaphoreType.DMA,
            pltpu.SemaphoreType.DMA,
        ],
    )
    def k(table_hbm, textt_hbm, out_hbm, idx_v, acc_v, sem, idx_sem):
        wid = lax.axis_index("s") * _NC + lax.axis_index("c")
        base = wid * b_per_w
        idx_cp = pltpu.async_copy(
            textt_hbm.at[:, pl.ds(base, b_per_w)], idx_v, idx_sem)

        zero = jnp.zeros((_L,), jnp.float32)

        def zloop(r, carry):
            for s in range(n_slice):
                acc_v[r, pl.ds(s * _L, _L)] = zero
            return carry

        lax.fori_loop(0, b_per_w, zloop, 0)
        idx_cp.wait()

        def fire(j, carry):
            pltpu.async_copy(table_hbm.at[idx_v.at[j]], acc_v, sem, add=True)
            return carry

        lax.fori_loop(0, hist, fire, 0)

        def drain(j, carry):
            pltpu.make_async_copy(table_hbm.at[idx_v.at[0]], acc_v, sem).wait()
            return carry

        lax.fori_loop(0, hist, drain, 0)
        pltpu.sync_copy(acc_v, out_hbm.at[pl.ds(base, b_per_w)])

    return k(emb_table, text_t)


def _head_body(x_ref, w_ref, b_ref, o_ref, *, inv_len):
    x = jnp.tanh(x_ref[...] * inv_len)
    logits = lax.dot_general(
        w_ref[...], x, dimension_numbers=(((1,), (1,)), ((), ())),
        preferred_element_type=jnp.float32)
    logits = logits + b_ref[...]
    m = jnp.max(logits, axis=0, keepdims=True)
    e = jnp.exp(logits - m)
    o_ref[...] = e / jnp.sum(e, axis=0, keepdims=True)


def kernel(text, offsets, emb_table, fc_w, fc_b):
    del offsets
    batch, hist = text.shape
    num_class = fc_w.shape[0]
    text_t = text.astype(jnp.int32).T
    pooled = _pooled_sum(emb_table, text_t)
    head = pl.pallas_call(
        functools.partial(_head_body, inv_len=1.0 / hist),
        out_shape=jax.ShapeDtypeStruct((num_class, batch), jnp.float32),
    )
    return head(pooled, fc_w, fc_b.reshape(num_class, 1)).T
